# baseline (speedup 1.0000x reference)
import functools

import jax
import jax.numpy as jnp
from jax import lax
from jax.experimental import pallas as pl
from jax.experimental.pallas import tpu as pltpu
from jax.experimental.pallas import tpu_sc as plsc

_N_ENT = 1000000
_EMB = 64
_BATCH = 16384
_L = 16
_NC, _NS = 2, 16
_NW = _NC * _NS
_B_PER_W = _BATCH // _NW
_CHUNK = 128
_NCHUNK = _B_PER_W // _CHUNK
_GROUPS = _CHUNK // _L
_TBLK = 2048


def _tc_transpose_body(src_ref, out_ref):
    ii = lax.broadcasted_iota(jnp.int32, (_EMB, _EMB), 0)
    jj = lax.broadcasted_iota(jnp.int32, (_EMB, _EMB), 1)
    eye = jnp.where(ii == jj, 1.0, 0.0).astype(jnp.float32)
    for k in range(_TBLK // 128):
        sub = src_ref[:, pl.ds(k * 128, 128)]
        t = lax.dot_general(sub, eye, (((0,), (0,)), ((), ())),
                            preferred_element_type=jnp.float32)
        out_ref[pl.ds(k * 128, 128), 0:_EMB] = t


def _tc_transpose(ent_t):
    grid = (pl.cdiv(_N_ENT, _TBLK),)
    return pl.pallas_call(
        _tc_transpose_body,
        grid=grid,
        in_specs=[pl.BlockSpec((_EMB, _TBLK), lambda i: (0, i))],
        out_specs=pl.BlockSpec((_TBLK, 2 * _EMB), lambda i: (i, 0)),
        out_shape=jax.ShapeDtypeStruct((_N_ENT, 2 * _EMB), jnp.float32),
    )(ent_t)


def _sc_body(head_hbm, rel_hbm, tail_hbm, ent_hbm, relemb_hbm, out_hbm,
             idx_h, idx_r, idx_t, h_buf, r_buf, t_buf, out_v, scr, sem):
    wid = lax.axis_index("s") * _NC + lax.axis_index("c")
    base = wid * _B_PER_W

    for j in range(_NCHUNK):
        off = base + j * _CHUNK
        pltpu.sync_copy(head_hbm.at[pl.ds(off, _CHUNK)], idx_h.at[j])
        pltpu.sync_copy(rel_hbm.at[pl.ds(off, _CHUNK)], idx_r.at[j])
        pltpu.sync_copy(tail_hbm.at[pl.ds(off, _CHUNK)], idx_t.at[j])

    iota = lax.iota(jnp.int32, _L)

    for j in range(_NCHUNK):
        ch = pltpu.async_copy(ent_hbm.at[idx_h.at[j]], h_buf, sem)
        cr = pltpu.async_copy(relemb_hbm.at[idx_r.at[j]], r_buf, sem)
        ct = pltpu.async_copy(ent_hbm.at[idx_t.at[j]], t_buf, sem)
        ch.wait()
        cr.wait()
        ct.wait()

        def group(g, _):
            acc = jnp.zeros((_L,), jnp.float32)
            for i in range(_L):
                row = g * _L + i
                p = (h_buf[row, pl.ds(0, _L)]
                     * r_buf[row, pl.ds(0, _L)]
                     * t_buf[row, pl.ds(0, _L)])
                for c in range(1, _EMB // _L):
                    s = pl.ds(c * _L, _L)
                    p = p + h_buf[row, s] * r_buf[row, s] * t_buf[row, s]
                scr[pl.ds(0, _L)] = p
                q = p + scr[pl.ds(8, _L)]
                s0 = ((q[0] + q[1]) + (q[2] + q[3])) \
                    + ((q[4] + q[5]) + (q[6] + q[7]))
                acc = jnp.where(iota == i, s0, acc)
            out_v[pl.ds(j * _CHUNK + g * _L, _L)] = acc
            return 0

        lax.fori_loop(0, _GROUPS, group, 0)

    pltpu.sync_copy(out_v, out_hbm.at[pl.ds(base, _B_PER_W)])


def kernel(head, relation, tail, entity_emb, relation_emb):
    head = head.astype(jnp.int32)
    relation = relation.astype(jnp.int32)
    tail = tail.astype(jnp.int32)
    ent_rm = _tc_transpose(entity_emb.T)
    mesh = plsc.VectorSubcoreMesh(core_axis_name="c", subcore_axis_name="s",
                                  num_cores=_NC)
    run = pl.kernel(
        _sc_body,
        mesh=mesh,
        compiler_params=pltpu.CompilerParams(use_tc_tiling_on_sc=False),
        out_type=jax.ShapeDtypeStruct((_BATCH,), jnp.float32),
        scratch_types=[
            pltpu.VMEM((_NCHUNK, _CHUNK), jnp.int32),
            pltpu.VMEM((_NCHUNK, _CHUNK), jnp.int32),
            pltpu.VMEM((_NCHUNK, _CHUNK), jnp.int32),
            pltpu.VMEM((_CHUNK, 2 * _EMB), jnp.float32),
            pltpu.VMEM((_CHUNK, _EMB), jnp.float32),
            pltpu.VMEM((_CHUNK, 2 * _EMB), jnp.float32),
            pltpu.VMEM((_B_PER_W,), jnp.float32),
            pltpu.VMEM((_L + 8,), jnp.float32),
            pltpu.SemaphoreType.DMA,
        ],
    )
    return run(head, relation, tail, ent_rm, relation_emb)


# speedup vs baseline: 1.7234x; 1.7234x over previous
import functools

import jax
import jax.numpy as jnp
from jax import lax
from jax.experimental import pallas as pl
from jax.experimental.pallas import tpu as pltpu
from jax.experimental.pallas import tpu_sc as plsc

_N_ENT = 1000000
_EMB = 64
_BATCH = 16384
_L = 16
_NC, _NS = 2, 16
_NW = _NC * _NS
_B_PER_W = _BATCH // _NW
_CHUNK = 128
_NCHUNK = _B_PER_W // _CHUNK
_GROUPS = _CHUNK // _L
_TBLK = 8192


def _tc_transpose_body(src_ref, out_ref):
    out_ref[:, 0:_EMB] = src_ref[...].T


def _tc_transpose(ent_t):
    grid = (pl.cdiv(_N_ENT, _TBLK),)
    return pl.pallas_call(
        _tc_transpose_body,
        grid=grid,
        in_specs=[pl.BlockSpec((_EMB, _TBLK), lambda i: (0, i))],
        out_specs=pl.BlockSpec((_TBLK, 2 * _EMB), lambda i: (i, 0)),
        out_shape=jax.ShapeDtypeStruct((_N_ENT, 2 * _EMB), jnp.float32),
    )(ent_t)


def _sc_body(head_hbm, rel_hbm, tail_hbm, ent_hbm, relemb_hbm, out_hbm,
             idx_h, idx_r, idx_t, h_buf, r_buf, t_buf, out_v, scr, sem):
    wid = lax.axis_index("s") * _NC + lax.axis_index("c")
    base = wid * _B_PER_W

    for j in range(_NCHUNK):
        off = base + j * _CHUNK
        pltpu.sync_copy(head_hbm.at[pl.ds(off, _CHUNK)], idx_h.at[j])
        pltpu.sync_copy(rel_hbm.at[pl.ds(off, _CHUNK)], idx_r.at[j])
        pltpu.sync_copy(tail_hbm.at[pl.ds(off, _CHUNK)], idx_t.at[j])

    iota = lax.iota(jnp.int32, _L)

    for j in range(_NCHUNK):
        ch = pltpu.async_copy(ent_hbm.at[idx_h.at[j]], h_buf, sem)
        cr = pltpu.async_copy(relemb_hbm.at[idx_r.at[j]], r_buf, sem)
        ct = pltpu.async_copy(ent_hbm.at[idx_t.at[j]], t_buf, sem)
        ch.wait()
        cr.wait()
        ct.wait()

        def group(g, _):
            acc = jnp.zeros((_L,), jnp.float32)
            for i in range(_L):
                row = g * _L + i
                p = (h_buf[row, pl.ds(0, _L)]
                     * r_buf[row, pl.ds(0, _L)]
                     * t_buf[row, pl.ds(0, _L)])
                for c in range(1, _EMB // _L):
                    s = pl.ds(c * _L, _L)
                    p = p + h_buf[row, s] * r_buf[row, s] * t_buf[row, s]
                scr[pl.ds(0, _L)] = p
                q = p + scr[pl.ds(8, _L)]
                s0 = ((q[0] + q[1]) + (q[2] + q[3])) \
                    + ((q[4] + q[5]) + (q[6] + q[7]))
                acc = jnp.where(iota == i, s0, acc)
            out_v[pl.ds(j * _CHUNK + g * _L, _L)] = acc
            return 0

        lax.fori_loop(0, _GROUPS, group, 0)

    pltpu.sync_copy(out_v, out_hbm.at[pl.ds(base, _B_PER_W)])


def kernel(head, relation, tail, entity_emb, relation_emb):
    head = head.astype(jnp.int32)
    relation = relation.astype(jnp.int32)
    tail = tail.astype(jnp.int32)
    ent_rm = _tc_transpose(entity_emb.T)
    mesh = plsc.VectorSubcoreMesh(core_axis_name="c", subcore_axis_name="s",
                                  num_cores=_NC)
    run = pl.kernel(
        _sc_body,
        mesh=mesh,
        compiler_params=pltpu.CompilerParams(use_tc_tiling_on_sc=False),
        out_type=jax.ShapeDtypeStruct((_BATCH,), jnp.float32),
        scratch_types=[
            pltpu.VMEM((_NCHUNK, _CHUNK), jnp.int32),
            pltpu.VMEM((_NCHUNK, _CHUNK), jnp.int32),
            pltpu.VMEM((_NCHUNK, _CHUNK), jnp.int32),
            pltpu.VMEM((_CHUNK, 2 * _EMB), jnp.float32),
            pltpu.VMEM((_CHUNK, _EMB), jnp.float32),
            pltpu.VMEM((_CHUNK, 2 * _EMB), jnp.float32),
            pltpu.VMEM((_B_PER_W,), jnp.float32),
            pltpu.VMEM((_L + 8,), jnp.float32),
            pltpu.SemaphoreType.DMA,
        ],
    )
    return run(head, relation, tail, ent_rm, relation_emb)


# speedup vs baseline: 1.8371x; 1.0660x over previous
import functools

import jax
import jax.numpy as jnp
from jax import lax
from jax.experimental import pallas as pl
from jax.experimental.pallas import tpu as pltpu
from jax.experimental.pallas import tpu_sc as plsc

_N_ENT = 1000000
_EMB = 64
_BATCH = 16384
_L = 16
_NC, _NS = 2, 16
_NW = _NC * _NS
_B_PER_W = _BATCH // _NW
_CHUNK = 128
_NCHUNK = _B_PER_W // _CHUNK
_GROUPS = _CHUNK // _L
_TBLK = 16384


def _tc_transpose_body(src_ref, out_ref):
    out_ref[:, 0:_EMB] = src_ref[...].T


def _tc_transpose(ent_t):
    grid = (pl.cdiv(_N_ENT, _TBLK),)
    return pl.pallas_call(
        _tc_transpose_body,
        grid=grid,
        in_specs=[pl.BlockSpec((_EMB, _TBLK), lambda i: (0, i))],
        out_specs=pl.BlockSpec((_TBLK, 2 * _EMB), lambda i: (i, 0)),
        out_shape=jax.ShapeDtypeStruct((_N_ENT, 2 * _EMB), jnp.float32),
    )(ent_t)


def _sc_body(head_hbm, rel_hbm, tail_hbm, ent_hbm, relemb_hbm, out_hbm,
             idx_h, idx_r, idx_t, h_buf, r_buf, t_buf, out_v, scr, sem):
    wid = lax.axis_index("s") * _NC + lax.axis_index("c")
    base = wid * _B_PER_W

    for j in range(_NCHUNK):
        off = base + j * _CHUNK
        pltpu.sync_copy(head_hbm.at[pl.ds(off, _CHUNK)], idx_h.at[j])
        pltpu.sync_copy(rel_hbm.at[pl.ds(off, _CHUNK)], idx_r.at[j])
        pltpu.sync_copy(tail_hbm.at[pl.ds(off, _CHUNK)], idx_t.at[j])

    iota = lax.iota(jnp.int32, _L)

    for j in range(_NCHUNK):
        ch = pltpu.async_copy(ent_hbm.at[idx_h.at[j]], h_buf, sem)
        cr = pltpu.async_copy(relemb_hbm.at[idx_r.at[j]], r_buf, sem)
        ct = pltpu.async_copy(ent_hbm.at[idx_t.at[j]], t_buf, sem)
        ch.wait()
        cr.wait()
        ct.wait()

        def group(g, _):
            acc = jnp.zeros((_L,), jnp.float32)
            for i in range(_L):
                row = g * _L + i
                p = (h_buf[row, pl.ds(0, _L)]
                     * r_buf[row, pl.ds(0, _L)]
                     * t_buf[row, pl.ds(0, _L)])
                for c in range(1, _EMB // _L):
                    s = pl.ds(c * _L, _L)
                    p = p + h_buf[row, s] * r_buf[row, s] * t_buf[row, s]
                scr[pl.ds(0, _L)] = p
                q = p + scr[pl.ds(8, _L)]
                s0 = ((q[0] + q[1]) + (q[2] + q[3])) \
                    + ((q[4] + q[5]) + (q[6] + q[7]))
                acc = jnp.where(iota == i, s0, acc)
            out_v[pl.ds(j * _CHUNK + g * _L, _L)] = acc
            return 0

        lax.fori_loop(0, _GROUPS, group, 0)

    pltpu.sync_copy(out_v, out_hbm.at[pl.ds(base, _B_PER_W)])


def kernel(head, relation, tail, entity_emb, relation_emb):
    head = head.astype(jnp.int32)
    relation = relation.astype(jnp.int32)
    tail = tail.astype(jnp.int32)
    ent_rm = _tc_transpose(entity_emb.T)
    mesh = plsc.VectorSubcoreMesh(core_axis_name="c", subcore_axis_name="s",
                                  num_cores=_NC)
    run = pl.kernel(
        _sc_body,
        mesh=mesh,
        compiler_params=pltpu.CompilerParams(use_tc_tiling_on_sc=False),
        out_type=jax.ShapeDtypeStruct((_BATCH,), jnp.float32),
        scratch_types=[
            pltpu.VMEM((_NCHUNK, _CHUNK), jnp.int32),
            pltpu.VMEM((_NCHUNK, _CHUNK), jnp.int32),
            pltpu.VMEM((_NCHUNK, _CHUNK), jnp.int32),
            pltpu.VMEM((_CHUNK, 2 * _EMB), jnp.float32),
            pltpu.VMEM((_CHUNK, _EMB), jnp.float32),
            pltpu.VMEM((_CHUNK, 2 * _EMB), jnp.float32),
            pltpu.VMEM((_B_PER_W,), jnp.float32),
            pltpu.VMEM((_L + 8,), jnp.float32),
            pltpu.SemaphoreType.DMA,
        ],
    )
    return run(head, relation, tail, ent_rm, relation_emb)


# speedup vs baseline: 1.8436x; 1.0035x over previous
import functools

import jax
import jax.numpy as jnp
from jax import lax
from jax.experimental import pallas as pl
from jax.experimental.pallas import tpu as pltpu
from jax.experimental.pallas import tpu_sc as plsc

_N_ENT = 1000000
_EMB = 64
_BATCH = 16384
_L = 16
_NC, _NS = 2, 16
_NW = _NC * _NS
_B_PER_W = _BATCH // _NW
_CHUNK = 128
_NCHUNK = _B_PER_W // _CHUNK
_GROUPS = _CHUNK // _L
_TBLK = 8192
_SPLIT = 524288
_NBLK2 = _SPLIT // _TBLK
_LASTB = pl.cdiv(_N_ENT, _TBLK) - 1


def _tc_transpose_body(src1_ref, src2_ref, out_ref):
    out_ref[:, 0:_EMB] = src1_ref[...].T
    out_ref[:, _EMB:2 * _EMB] = src2_ref[...].T


def _tc_transpose(ent_t):
    return pl.pallas_call(
        _tc_transpose_body,
        grid=(_NBLK2,),
        in_specs=[
            pl.BlockSpec((_EMB, _TBLK), lambda i: (0, i)),
            pl.BlockSpec((_EMB, _TBLK),
                         lambda i: (0, jnp.minimum(i + _NBLK2, _LASTB))),
        ],
        out_specs=pl.BlockSpec((_TBLK, 2 * _EMB), lambda i: (i, 0)),
        out_shape=jax.ShapeDtypeStruct((_SPLIT, 2 * _EMB), jnp.float32),
    )(ent_t, ent_t)


def _sc_body(head_hbm, rel_hbm, tail_hbm, ent_hbm, relemb_hbm, out_hbm,
             idx_h, idx_r, idx_t, par_h, par_t,
             h_buf, r_buf, t_buf, out_v, scr, sem):
    wid = lax.axis_index("s") * _NC + lax.axis_index("c")
    base = wid * _B_PER_W

    for j in range(_NCHUNK):
        off = base + j * _CHUNK
        pltpu.sync_copy(head_hbm.at[pl.ds(off, _CHUNK)], idx_h.at[j])
        pltpu.sync_copy(rel_hbm.at[pl.ds(off, _CHUNK)], idx_r.at[j])
        pltpu.sync_copy(tail_hbm.at[pl.ds(off, _CHUNK)], idx_t.at[j])
    for j in range(_NCHUNK):
        for k in range(_CHUNK // _L):
            s = pl.ds(k * _L, _L)
            eh = idx_h[j, s]
            et = idx_t[j, s]
            hi_h = eh >= _SPLIT
            hi_t = et >= _SPLIT
            par_h[j, s] = jnp.where(hi_h, _EMB, 0)
            par_t[j, s] = jnp.where(hi_t, _EMB, 0)
            idx_h[j, s] = jnp.where(hi_h, eh - _SPLIT, eh)
            idx_t[j, s] = jnp.where(hi_t, et - _SPLIT, et)

    iota = lax.iota(jnp.int32, _L)

    for j in range(_NCHUNK):
        ch = pltpu.async_copy(ent_hbm.at[idx_h.at[j]], h_buf, sem)
        cr = pltpu.async_copy(relemb_hbm.at[idx_r.at[j]], r_buf, sem)
        ct = pltpu.async_copy(ent_hbm.at[idx_t.at[j]], t_buf, sem)
        ch.wait()
        cr.wait()
        ct.wait()

        def group(g, _):
            acc = jnp.zeros((_L,), jnp.float32)
            gs = pl.ds(g * _L, _L)
            bh = par_h[j, gs]
            bt = par_t[j, gs]
            for i in range(_L):
                row = g * _L + i
                bhi = bh[i]
                bti = bt[i]
                p = (h_buf[row, pl.ds(bhi, _L)]
                     * r_buf[row, pl.ds(0, _L)]
                     * t_buf[row, pl.ds(bti, _L)])
                for c in range(1, _EMB // _L):
                    p = p + (h_buf[row, pl.ds(bhi + c * _L, _L)]
                             * r_buf[row, pl.ds(c * _L, _L)]
                             * t_buf[row, pl.ds(bti + c * _L, _L)])
                scr[pl.ds(0, _L)] = p
                q = p + scr[pl.ds(8, _L)]
                s0 = ((q[0] + q[1]) + (q[2] + q[3])) \
                    + ((q[4] + q[5]) + (q[6] + q[7]))
                acc = jnp.where(iota == i, s0, acc)
            out_v[pl.ds(j * _CHUNK + g * _L, _L)] = acc
            return 0

        lax.fori_loop(0, _GROUPS, group, 0)

    pltpu.sync_copy(out_v, out_hbm.at[pl.ds(base, _B_PER_W)])


def kernel(head, relation, tail, entity_emb, relation_emb):
    head = head.astype(jnp.int32)
    relation = relation.astype(jnp.int32)
    tail = tail.astype(jnp.int32)
    ent_rm = _tc_transpose(entity_emb.T)
    mesh = plsc.VectorSubcoreMesh(core_axis_name="c", subcore_axis_name="s",
                                  num_cores=_NC)
    run = pl.kernel(
        _sc_body,
        mesh=mesh,
        compiler_params=pltpu.CompilerParams(use_tc_tiling_on_sc=False),
        out_type=jax.ShapeDtypeStruct((_BATCH,), jnp.float32),
        scratch_types=[
            pltpu.VMEM((_NCHUNK, _CHUNK), jnp.int32),
            pltpu.VMEM((_NCHUNK, _CHUNK), jnp.int32),
            pltpu.VMEM((_NCHUNK, _CHUNK), jnp.int32),
            pltpu.VMEM((_NCHUNK, _CHUNK), jnp.int32),
            pltpu.VMEM((_NCHUNK, _CHUNK), jnp.int32),
            pltpu.VMEM((_CHUNK, 2 * _EMB), jnp.float32),
            pltpu.VMEM((_CHUNK, _EMB), jnp.float32),
            pltpu.VMEM((_CHUNK, 2 * _EMB), jnp.float32),
            pltpu.VMEM((_B_PER_W,), jnp.float32),
            pltpu.VMEM((_L + 8,), jnp.float32),
            pltpu.SemaphoreType.DMA,
        ],
    )
    return run(head, relation, tail, ent_rm, relation_emb)


# speedup vs baseline: 1.9411x; 1.0529x over previous
import functools

import jax
import jax.numpy as jnp
from jax import lax
from jax.experimental import pallas as pl
from jax.experimental.pallas import tpu as pltpu
from jax.experimental.pallas import tpu_sc as plsc

_N_ENT = 1000000
_EMB = 64
_BATCH = 16384
_L = 16
_NC, _NS = 2, 16
_NW = _NC * _NS
_B_PER_W = _BATCH // _NW
_CHUNK = 128
_NCHUNK = _B_PER_W // _CHUNK
_GROUPS = _CHUNK // _L
_TBLK = 16384
_SPLIT = 524288
_NBLK2 = _SPLIT // _TBLK
_LASTB = pl.cdiv(_N_ENT, _TBLK) - 1


def _tc_transpose_body(src1_ref, src2_ref, out_ref):
    out_ref[:, 0:_EMB] = src1_ref[...].T
    out_ref[:, _EMB:2 * _EMB] = src2_ref[...].T


def _tc_transpose(ent_t):
    return pl.pallas_call(
        _tc_transpose_body,
        grid=(_NBLK2,),
        in_specs=[
            pl.BlockSpec((_EMB, _TBLK), lambda i: (0, i)),
            pl.BlockSpec((_EMB, _TBLK),
                         lambda i: (0, jnp.minimum(i + _NBLK2, _LASTB))),
        ],
        out_specs=pl.BlockSpec((_TBLK, 2 * _EMB), lambda i: (i, 0)),
        out_shape=jax.ShapeDtypeStruct((_SPLIT, 2 * _EMB), jnp.float32),
    )(ent_t, ent_t)


def _sc_body(head_hbm, rel_hbm, tail_hbm, ent_hbm, relemb_hbm, out_hbm,
             idx_h, idx_r, idx_t, par_h, par_t,
             h_buf, r_buf, t_buf, out_v, scr, sem):
    wid = lax.axis_index("s") * _NC + lax.axis_index("c")
    base = wid * _B_PER_W

    for j in range(_NCHUNK):
        off = base + j * _CHUNK
        pltpu.sync_copy(head_hbm.at[pl.ds(off, _CHUNK)], idx_h.at[j])
        pltpu.sync_copy(rel_hbm.at[pl.ds(off, _CHUNK)], idx_r.at[j])
        pltpu.sync_copy(tail_hbm.at[pl.ds(off, _CHUNK)], idx_t.at[j])
    for j in range(_NCHUNK):
        for k in range(_CHUNK // _L):
            s = pl.ds(k * _L, _L)
            eh = idx_h[j, s]
            et = idx_t[j, s]
            hi_h = eh >= _SPLIT
            hi_t = et >= _SPLIT
            par_h[j, s] = jnp.where(hi_h, _EMB, 0)
            par_t[j, s] = jnp.where(hi_t, _EMB, 0)
            idx_h[j, s] = jnp.where(hi_h, eh - _SPLIT, eh)
            idx_t[j, s] = jnp.where(hi_t, et - _SPLIT, et)

    iota = lax.iota(jnp.int32, _L)

    for j in range(_NCHUNK):
        ch = pltpu.async_copy(ent_hbm.at[idx_h.at[j]], h_buf, sem)
        cr = pltpu.async_copy(relemb_hbm.at[idx_r.at[j]], r_buf, sem)
        ct = pltpu.async_copy(ent_hbm.at[idx_t.at[j]], t_buf, sem)
        ch.wait()
        cr.wait()
        ct.wait()

        def group(g, _):
            acc = jnp.zeros((_L,), jnp.float32)
            gs = pl.ds(g * _L, _L)
            bh = par_h[j, gs]
            bt = par_t[j, gs]
            for i in range(_L):
                row = g * _L + i
                bhi = bh[i]
                bti = bt[i]
                p = (h_buf[row, pl.ds(bhi, _L)]
                     * r_buf[row, pl.ds(0, _L)]
                     * t_buf[row, pl.ds(bti, _L)])
                for c in range(1, _EMB // _L):
                    p = p + (h_buf[row, pl.ds(bhi + c * _L, _L)]
                             * r_buf[row, pl.ds(c * _L, _L)]
                             * t_buf[row, pl.ds(bti + c * _L, _L)])
                scr[pl.ds(0, _L)] = p
                q = p + scr[pl.ds(8, _L)]
                s0 = ((q[0] + q[1]) + (q[2] + q[3])) \
                    + ((q[4] + q[5]) + (q[6] + q[7]))
                acc = jnp.where(iota == i, s0, acc)
            out_v[pl.ds(j * _CHUNK + g * _L, _L)] = acc
            return 0

        lax.fori_loop(0, _GROUPS, group, 0)

    pltpu.sync_copy(out_v, out_hbm.at[pl.ds(base, _B_PER_W)])


def kernel(head, relation, tail, entity_emb, relation_emb):
    head = head.astype(jnp.int32)
    relation = relation.astype(jnp.int32)
    tail = tail.astype(jnp.int32)
    ent_rm = _tc_transpose(entity_emb.T)
    mesh = plsc.VectorSubcoreMesh(core_axis_name="c", subcore_axis_name="s",
                                  num_cores=_NC)
    run = pl.kernel(
        _sc_body,
        mesh=mesh,
        compiler_params=pltpu.CompilerParams(use_tc_tiling_on_sc=False),
        out_type=jax.ShapeDtypeStruct((_BATCH,), jnp.float32),
        scratch_types=[
            pltpu.VMEM((_NCHUNK, _CHUNK), jnp.int32),
            pltpu.VMEM((_NCHUNK, _CHUNK), jnp.int32),
            pltpu.VMEM((_NCHUNK, _CHUNK), jnp.int32),
            pltpu.VMEM((_NCHUNK, _CHUNK), jnp.int32),
            pltpu.VMEM((_NCHUNK, _CHUNK), jnp.int32),
            pltpu.VMEM((_CHUNK, 2 * _EMB), jnp.float32),
            pltpu.VMEM((_CHUNK, _EMB), jnp.float32),
            pltpu.VMEM((_CHUNK, 2 * _EMB), jnp.float32),
            pltpu.VMEM((_B_PER_W,), jnp.float32),
            pltpu.VMEM((_L + 8,), jnp.float32),
            pltpu.SemaphoreType.DMA,
        ],
    )
    return run(head, relation, tail, ent_rm, relation_emb)
